# baseline (device time: 74959 ns/iter reference)
import jax
import jax.numpy as jnp
from jax import lax
from jax.experimental import pallas as pl
from jax.experimental.pallas import tpu as pltpu

N_DEV = 32
FP8 = jnp.bfloat16


def kernel(x, w_mat, scale_x, scale_w):
    m_per, k = x.shape
    _, n = w_mat.shape
    n_per = n // N_DEV
    print(f"[kernel] dtypes: x={x.dtype} w={w_mat.dtype} "
          f"sx={scale_x.dtype} shapes x={x.shape} w={w_mat.shape}")

    def body(x_ref, w_ref, sx_ref, sw_ref, out_ref,
             xbuf, wbuf, comm_ref, copy_sems, send_sems, recv_sems):
        me = lax.axis_index("i")
        s = sx_ref[0] * sw_ref[0]

        xbuf[...] = x_ref[...].astype(FP8)

        def w_copy(d, slot):
            j = lax.rem(me + d, N_DEV)
            return pltpu.make_async_copy(
                w_ref.at[:, pl.ds(j * n_per, n_per)],
                wbuf.at[slot],
                copy_sems.at[slot],
            )

        w_copy(0, 0).start()
        rdmas = []
        for d in range(N_DEV):
            slot = d % 2
            if d + 1 < N_DEV:
                w_copy(d + 1, 1 - slot).start()
            w_copy(d, slot).wait()
            chunk = jnp.dot(
                xbuf[...], wbuf[slot].astype(FP8),
                preferred_element_type=jnp.float32,
            ) * s
            if d == 0:
                out_ref[pl.ds(me * m_per, m_per), :] = chunk
            else:
                comm_ref[d] = chunk
                rdma = pltpu.make_async_remote_copy(
                    src_ref=comm_ref.at[d],
                    dst_ref=out_ref.at[pl.ds(me * m_per, m_per), :],
                    send_sem=send_sems.at[d],
                    recv_sem=recv_sems.at[d],
                    device_id=(lax.rem(me + d, N_DEV),),
                    device_id_type=pl.DeviceIdType.MESH,
                )
                rdma.start()
                rdmas.append(rdma)

        for rdma in rdmas:
            rdma.wait()

    return pl.pallas_call(
        body,
        out_shape=jax.ShapeDtypeStruct((N_DEV * m_per, n_per), jnp.float32),
        in_specs=[
            pl.BlockSpec(memory_space=pltpu.VMEM),
            pl.BlockSpec(memory_space=pltpu.MemorySpace.HBM),
            pl.BlockSpec(memory_space=pltpu.SMEM),
            pl.BlockSpec(memory_space=pltpu.SMEM),
        ],
        out_specs=pl.BlockSpec(memory_space=pltpu.VMEM),
        scratch_shapes=[
            pltpu.VMEM((m_per, k), FP8),
            pltpu.VMEM((2, k, n_per), w_mat.dtype),
            pltpu.VMEM((N_DEV, m_per, n_per), jnp.float32),
            pltpu.SemaphoreType.DMA((2,)),
            pltpu.SemaphoreType.DMA((N_DEV,)),
            pltpu.SemaphoreType.DMA((N_DEV,)),
        ],
    )(x, w_mat, scale_x, scale_w)


# device time: 67036 ns/iter; 1.1182x vs baseline; 1.1182x over previous
import jax
import jax.numpy as jnp
from jax import lax
from jax.experimental import pallas as pl
from jax.experimental.pallas import tpu as pltpu

N_DEV = 32
FP8 = jnp.float8_e4m3fn
COMM_DT = jnp.bfloat16


def kernel(x, w_mat, scale_x, scale_w):
    m_per, k = x.shape
    _, n = w_mat.shape
    n_per = n // N_DEV

    def body(x_ref, w_ref, sx_ref, sw_ref, out_ref,
             xbuf, wbuf, comm_ref, rbuf, copy_sems, send_sems, recv_sems):
        me = lax.axis_index("i")
        s = sx_ref[0] * sw_ref[0]

        xbuf[...] = x_ref[...].astype(FP8)

        def w_copy(d, slot):
            j = lax.rem(me + d, N_DEV)
            return pltpu.make_async_copy(
                w_ref.at[:, pl.ds(j * n_per, n_per)],
                wbuf.at[slot],
                copy_sems.at[slot],
            )

        w_copy(0, 0).start()
        rdmas = []
        for d in range(N_DEV):
            slot = d % 2
            if d + 1 < N_DEV:
                w_copy(d + 1, 1 - slot).start()
            w_copy(d, slot).wait()
            chunk = jnp.dot(
                xbuf[...], wbuf[slot].astype(FP8),
                preferred_element_type=jnp.float32,
            ) * s
            if d == 0:
                out_ref[pl.ds(me * m_per, m_per), :] = chunk
            else:
                comm_ref[d] = chunk.astype(COMM_DT)
                rdma = pltpu.make_async_remote_copy(
                    src_ref=comm_ref.at[d],
                    dst_ref=rbuf.at[d],
                    send_sem=send_sems.at[d],
                    recv_sem=recv_sems.at[d],
                    device_id=(lax.rem(me + d, N_DEV),),
                    device_id_type=pl.DeviceIdType.MESH,
                )
                rdma.start()
                rdmas.append(rdma)

        for d, rdma in zip(range(1, N_DEV), rdmas):
            rdma.wait()
            src = lax.rem(me - d + N_DEV, N_DEV)
            out_ref[pl.ds(src * m_per, m_per), :] = rbuf[d].astype(jnp.float32)

    return pl.pallas_call(
        body,
        out_shape=jax.ShapeDtypeStruct((N_DEV * m_per, n_per), jnp.float32),
        in_specs=[
            pl.BlockSpec(memory_space=pltpu.VMEM),
            pl.BlockSpec(memory_space=pltpu.MemorySpace.HBM),
            pl.BlockSpec(memory_space=pltpu.SMEM),
            pl.BlockSpec(memory_space=pltpu.SMEM),
        ],
        out_specs=pl.BlockSpec(memory_space=pltpu.VMEM),
        scratch_shapes=[
            pltpu.VMEM((m_per, k), FP8),
            pltpu.VMEM((2, k, n_per), w_mat.dtype),
            pltpu.VMEM((N_DEV, m_per, n_per), COMM_DT),
            pltpu.VMEM((N_DEV, m_per, n_per), COMM_DT),
            pltpu.SemaphoreType.DMA((2,)),
            pltpu.SemaphoreType.DMA((N_DEV,)),
            pltpu.SemaphoreType.DMA((N_DEV,)),
        ],
        compiler_params=pltpu.CompilerParams(
            vmem_limit_bytes=100 * 1024 * 1024),
    )(x, w_mat, scale_x, scale_w)


# device time: 64290 ns/iter; 1.1660x vs baseline; 1.0427x over previous
import jax
import jax.numpy as jnp
from jax import lax
from jax.experimental import pallas as pl
from jax.experimental.pallas import tpu as pltpu

N_DEV = 32
FP8 = jnp.float8_e4m3fn
COMM_DT = jnp.bfloat16


def kernel(x, w_mat, scale_x, scale_w):
    m_per, k = x.shape
    _, n = w_mat.shape
    n_per = n // N_DEV

    me_out = lax.axis_index("i")
    sched = lax.rem(me_out + jnp.arange(N_DEV, dtype=jnp.int32), N_DEV)

    def body(sched_ref, x_ref, w_ref, sx_ref, sw_ref, out_ref,
             xbuf, comm_ref, rbuf, send_sems, recv_sems):
        d = pl.program_id(0)
        me = lax.axis_index("i")
        s = sx_ref[0] * sw_ref[0]

        @pl.when(d == 0)
        def _():
            xbuf[...] = x_ref[...].astype(FP8)
            barrier = pltpu.get_barrier_semaphore()
            for p in range(1, N_DEV):
                pl.semaphore_signal(
                    barrier, inc=1,
                    device_id=(lax.rem(me + p, N_DEV),),
                    device_id_type=pl.DeviceIdType.MESH,
                )
            pl.semaphore_wait(barrier, N_DEV - 1)

        chunk = jnp.dot(
            xbuf[...], w_ref[...].astype(FP8),
            preferred_element_type=jnp.float32,
        ) * s

        @pl.when(d == 0)
        def _():
            out_ref[pl.ds(me * m_per, m_per), :] = chunk

        @pl.when(d != 0)
        def _():
            comm_ref[pl.ds(d * m_per, m_per), :] = chunk.astype(COMM_DT)
            rdma = pltpu.make_async_remote_copy(
                src_ref=comm_ref.at[pl.ds(d * m_per, m_per), :],
                dst_ref=rbuf.at[pl.ds(d * m_per, m_per), :],
                send_sem=send_sems.at[d],
                recv_sem=recv_sems.at[d],
                device_id=(sched_ref[d],),
                device_id_type=pl.DeviceIdType.MESH,
            )
            rdma.start()

        @pl.when(d == N_DEV - 1)
        def _():
            for dd in range(1, N_DEV):
                rdma = pltpu.make_async_remote_copy(
                    src_ref=comm_ref.at[pl.ds(dd * m_per, m_per), :],
                    dst_ref=rbuf.at[pl.ds(dd * m_per, m_per), :],
                    send_sem=send_sems.at[dd],
                    recv_sem=recv_sems.at[dd],
                    device_id=(sched_ref[dd],),
                    device_id_type=pl.DeviceIdType.MESH,
                )
                rdma.wait()
                src = lax.rem(me - dd + N_DEV, N_DEV)
                out_ref[pl.ds(src * m_per, m_per), :] = (
                    rbuf[pl.ds(dd * m_per, m_per), :].astype(jnp.float32))

    grid_spec = pltpu.PrefetchScalarGridSpec(
        num_scalar_prefetch=1,
        grid=(N_DEV,),
        in_specs=[
            pl.BlockSpec((m_per, k), lambda d, sched: (0, 0)),
            pl.BlockSpec((k, n_per), lambda d, sched: (0, sched[d])),
            pl.BlockSpec(memory_space=pltpu.SMEM),
            pl.BlockSpec(memory_space=pltpu.SMEM),
        ],
        out_specs=pl.BlockSpec((N_DEV * m_per, n_per), lambda d, sched: (0, 0)),
        scratch_shapes=[
            pltpu.VMEM((m_per, k), FP8),
            pltpu.VMEM((N_DEV * m_per, n_per), COMM_DT),
            pltpu.VMEM((N_DEV * m_per, n_per), COMM_DT),
            pltpu.SemaphoreType.DMA((N_DEV,)),
            pltpu.SemaphoreType.DMA((N_DEV,)),
        ],
    )

    return pl.pallas_call(
        body,
        grid_spec=grid_spec,
        out_shape=jax.ShapeDtypeStruct((N_DEV * m_per, n_per), jnp.float32),
        compiler_params=pltpu.CompilerParams(
            dimension_semantics=("arbitrary",),
            collective_id=1,
            vmem_limit_bytes=100 * 1024 * 1024),
    )(sched, x, w_mat, scale_x, scale_w)
